# baseline (speedup 1.0000x reference)
import jax
import jax.numpy as jnp
from jax import lax
from jax.experimental import pallas as pl
from jax.experimental.pallas import tpu as pltpu
from jax.experimental.pallas import tpu_sc as plsc

N_NODES = 10000
N_EDGES = 320000
D_NODE = 128
D_EDGE = 16
D_OUT = 128

LANES = 16
CW = 16
NC = 2
NS = 16
NW = NC * NS
EPW = N_EDGES // NW
CHUNK = 40
NCHUNKS = EPW // CHUNK
NPAIRS = NCHUNKS // 2
N_PAD = 10240
ROWS_PER_TILE = N_PAD // NS

_DOT = (((1,), (0,)), ((), ()))


def _pq_body(x_ref, ws_ref, wd_ref, p_ref, q_ref):
    x = x_ref[...]
    p_ref[...] = lax.dot_general(x, ws_ref[...], _DOT,
                                 preferred_element_type=jnp.float32)
    q_ref[...] = lax.dot_general(x, wd_ref[...], _DOT,
                                 preferred_element_type=jnp.float32)


_BN = 2000
_pq_call = pl.pallas_call(
    _pq_body,
    grid=(N_NODES // _BN,),
    in_specs=[
        pl.BlockSpec((_BN, D_NODE), lambda i: (i, 0)),
        pl.BlockSpec((D_NODE, D_OUT), lambda i: (0, 0)),
        pl.BlockSpec((D_NODE, D_OUT), lambda i: (0, 0)),
    ],
    out_specs=[
        pl.BlockSpec((_BN, D_OUT), lambda i: (i, 0)),
        pl.BlockSpec((_BN, D_OUT), lambda i: (i, 0)),
    ],
    out_shape=[
        jax.ShapeDtypeStruct((N_NODES, D_OUT), jnp.float32),
        jax.ShapeDtypeStruct((N_NODES, D_OUT), jnp.float32),
    ],
)


def _e1_body(ef_ref, we_ref, b1_ref, e1_ref):
    e1_ref[...] = lax.dot_general(ef_ref[...], we_ref[...], _DOT,
                                  preferred_element_type=jnp.float32) + b1_ref[...]


_BE = 8000
_e1_call = pl.pallas_call(
    _e1_body,
    grid=(N_EDGES // _BE,),
    in_specs=[
        pl.BlockSpec((_BE, D_EDGE), lambda i: (i, 0)),
        pl.BlockSpec((D_EDGE, D_OUT), lambda i: (0, 0)),
        pl.BlockSpec((1, D_OUT), lambda i: (0, 0)),
    ],
    out_specs=pl.BlockSpec((_BE, D_OUT), lambda i: (i, 0)),
    out_shape=jax.ShapeDtypeStruct((N_EDGES, D_OUT), jnp.float32),
)


def _sc_edge_body(p_hbm, q_hbm, e1_hbm, src_hbm, dst_hbm,
                  agg_out, cnt_out,
                  idx_s_a, idx_d_a, bp_a, bq_a, be_a,
                  idx_s_b, idx_d_b, bp_b, bq_b, be_b,
                  ones_v, zc_v,
                  agg_sh, cnt_sh,
                  sem_g_a, sem_i_a, sem_g_b, sem_i_b):
    cid = lax.axis_index("c")
    sid = lax.axis_index("s")
    wid = sid * NC + cid
    ebase = wid * EPW

    set_a = (idx_s_a, idx_d_a, bp_a, bq_a, be_a, sem_g_a, sem_i_a)
    set_b = (idx_s_b, idx_d_b, bp_b, bq_b, be_b, sem_g_b, sem_i_b)

    zf = jnp.zeros((LANES,), jnp.float32)
    onef = jnp.ones((LANES,), jnp.float32)

    def _zfill(r, carry):
        for c in range(D_OUT // LANES):
            be_a[r, pl.ds(c * LANES, LANES)] = zf
        zc_v[r, pl.ds(0, LANES)] = zf
        ones_v[r, pl.ds(0, LANES)] = onef
        return carry

    lax.fori_loop(0, CHUNK, _zfill, 0)

    zi = jnp.zeros((LANES,), jnp.int32)
    for _s in (set_a, set_b):
        for _r in (_s[0], _s[1]):
            _r[pl.ds(0, LANES)] = zi
            _r[pl.ds(16, LANES)] = zi
            _r[pl.ds(24, LANES)] = zi

    base_row = pl.multiple_of(sid * ROWS_PER_TILE, 8)
    for k in range(ROWS_PER_TILE // CHUNK):
        pltpu.sync_copy(be_a, agg_sh.at[pl.ds(base_row + k * CHUNK, CHUNK)])
        pltpu.sync_copy(zc_v, cnt_sh.at[pl.ds(base_row + k * CHUNK, CHUNK)])
    plsc.subcore_barrier()

    def _off(c):
        return pl.multiple_of(ebase + c * CHUNK, CHUNK)

    def issue_idx(c, s):
        pass

    def wait_idx(s):
        pass

    def issue_gathers(c, s):
        idx_s, idx_d, bp, bq, be, sem_g, _ = s
        off = _off(c)
        pltpu.async_copy(e1_hbm.at[pl.ds(off, CHUNK)], be, sem_g)
        pltpu.async_copy(p_hbm.at[idx_s], bp, sem_g)
        pltpu.async_copy(q_hbm.at[idx_d], bq, sem_g)

    def wait_gathers(s):
        idx_s, idx_d, bp, bq, be, sem_g, _ = s
        pltpu.make_async_copy(e1_hbm.at[pl.ds(0, CHUNK)], be, sem_g).wait()
        pltpu.make_async_copy(p_hbm.at[idx_s], bp, sem_g).wait()
        pltpu.make_async_copy(q_hbm.at[idx_d], bq, sem_g).wait()

    def compute(s):
        _, _, bp, bq, be, _, _ = s

        @plsc.parallel_loop(0, CHUNK, 1, unroll=2)
        def _row(r):
            for c in range(D_OUT // LANES):
                sl = pl.ds(c * LANES, LANES)
                be[r, sl] = jnp.maximum(bp[r, sl] + bq[r, sl] + be[r, sl], 0.0)

    def scatter(s):
        _, idx_d, _, _, be, _, _ = s
        pltpu.sync_copy(be, agg_sh.at[idx_d], add=True)
        pltpu.sync_copy(ones_v, cnt_sh.at[idx_d], add=True)

    issue_idx(0, set_a)
    wait_idx(set_a)
    issue_gathers(0, set_a)
    issue_idx(1, set_b)

    def _pair(ci, carry):
        for k, (s, t) in ((0, (set_a, set_b)), (1, (set_b, set_a))):
            c = 2 * ci + k
            wait_gathers(s)

            @pl.when(c + 1 < NCHUNKS)
            def _():
                wait_idx(t)
                issue_gathers(c + 1, t)

            compute(s)
            scatter(s)

            @pl.when(c + 2 < NCHUNKS)
            def _():
                issue_idx(c + 2, s)

        return carry

    lax.fori_loop(0, NPAIRS, _pair, 0)

    plsc.subcore_barrier()

    pltpu.sync_copy(agg_sh.at[pl.ds(base_row, ROWS_PER_TILE)],
                    agg_out.at[cid, pl.ds(base_row, ROWS_PER_TILE)])
    pltpu.sync_copy(cnt_sh.at[pl.ds(base_row, ROWS_PER_TILE)],
                    cnt_out.at[cid, pl.ds(base_row, ROWS_PER_TILE)])


_sc_edge = pl.kernel(
    _sc_edge_body,
    mesh=plsc.VectorSubcoreMesh(core_axis_name="c", subcore_axis_name="s"),
    compiler_params=pltpu.CompilerParams(use_tc_tiling_on_sc=False,
                                         needs_layout_passes=False),
    out_type=[
        jax.ShapeDtypeStruct((NC, N_PAD, D_OUT), jnp.float32),
        jax.ShapeDtypeStruct((NC, N_PAD, CW), jnp.float32),
    ],
    scratch_types=[
        pltpu.VMEM((CHUNK,), jnp.int32),
        pltpu.VMEM((CHUNK,), jnp.int32),
        pltpu.VMEM((CHUNK, D_OUT), jnp.float32),
        pltpu.VMEM((CHUNK, D_OUT), jnp.float32),
        pltpu.VMEM((CHUNK, D_OUT), jnp.float32),
        pltpu.VMEM((CHUNK,), jnp.int32),
        pltpu.VMEM((CHUNK,), jnp.int32),
        pltpu.VMEM((CHUNK, D_OUT), jnp.float32),
        pltpu.VMEM((CHUNK, D_OUT), jnp.float32),
        pltpu.VMEM((CHUNK, D_OUT), jnp.float32),
        pltpu.VMEM((CHUNK, CW), jnp.float32),
        pltpu.VMEM((CHUNK, CW), jnp.float32),
        pltpu.VMEM_SHARED((N_PAD, D_OUT), jnp.float32),
        pltpu.VMEM_SHARED((N_PAD, CW), jnp.float32),
        pltpu.SemaphoreType.DMA,
        pltpu.SemaphoreType.DMA,
        pltpu.SemaphoreType.DMA,
        pltpu.SemaphoreType.DMA,
    ],
)


def _out_body(a0_ref, a1_ref, c0_ref, c1_ref, w2_ref, b2_ref, o_ref):
    agg = a0_ref[...] + a1_ref[...]
    deg = c0_ref[...][:, :1] + c1_ref[...][:, :1]
    o_ref[...] = lax.dot_general(agg, w2_ref[...], _DOT,
                                 preferred_element_type=jnp.float32,
                                 precision=lax.Precision.HIGHEST) + deg * b2_ref[...]


_BO = 1000
_out_call = pl.pallas_call(
    _out_body,
    grid=(N_NODES // _BO,),
    in_specs=[
        pl.BlockSpec((_BO, D_OUT), lambda i: (i, 0)),
        pl.BlockSpec((_BO, D_OUT), lambda i: (i, 0)),
        pl.BlockSpec((_BO, CW), lambda i: (i, 0)),
        pl.BlockSpec((_BO, CW), lambda i: (i, 0)),
        pl.BlockSpec((D_OUT, D_OUT), lambda i: (0, 0)),
        pl.BlockSpec((1, D_OUT), lambda i: (0, 0)),
    ],
    out_specs=pl.BlockSpec((_BO, D_OUT), lambda i: (i, 0)),
    out_shape=jax.ShapeDtypeStruct((N_NODES, D_OUT), jnp.float32),
)


def kernel(node_feats, edge_index, edge_feats, W1, b1, W2, b2):
    src = edge_index[0].astype(jnp.int32)
    dst = edge_index[1].astype(jnp.int32)
    p, q = _pq_call(node_feats, W1[:D_NODE], W1[D_NODE:2 * D_NODE])
    e1 = _e1_call(edge_feats, W1[2 * D_NODE:], b1.reshape(1, D_OUT))
    agg2, cnt2 = _sc_edge(p, q, e1, src, dst)
    out = _out_call(agg2[0], agg2[1], cnt2[0], cnt2[1],
                    W2, b2.reshape(1, D_OUT))
    return out


# speedup vs baseline: 20.4940x; 20.4940x over previous
import jax
import jax.numpy as jnp
from jax import lax
from jax.experimental import pallas as pl
from jax.experimental.pallas import tpu as pltpu
from jax.experimental.pallas import tpu_sc as plsc

N_NODES = 10000
N_EDGES = 320000
D_NODE = 128
D_EDGE = 16
D_OUT = 128

LANES = 16
CW = 16
NC = 2
NS = 16
NW = NC * NS
EPW = N_EDGES // NW
CHUNK = 40
NCHUNKS = EPW // CHUNK
NPAIRS = NCHUNKS // 2
N_PAD = 10240
ROWS_PER_TILE = N_PAD // NS

_DOT = (((1,), (0,)), ((), ()))


def _pq_body(x_ref, ws_ref, wd_ref, p_ref, q_ref):
    x = x_ref[...]
    p_ref[...] = lax.dot_general(x, ws_ref[...], _DOT,
                                 preferred_element_type=jnp.float32)
    q_ref[...] = lax.dot_general(x, wd_ref[...], _DOT,
                                 preferred_element_type=jnp.float32)


_BN = 2000
_pq_call = pl.pallas_call(
    _pq_body,
    grid=(N_NODES // _BN,),
    in_specs=[
        pl.BlockSpec((_BN, D_NODE), lambda i: (i, 0)),
        pl.BlockSpec((D_NODE, D_OUT), lambda i: (0, 0)),
        pl.BlockSpec((D_NODE, D_OUT), lambda i: (0, 0)),
    ],
    out_specs=[
        pl.BlockSpec((_BN, D_OUT), lambda i: (i, 0)),
        pl.BlockSpec((_BN, D_OUT), lambda i: (i, 0)),
    ],
    out_shape=[
        jax.ShapeDtypeStruct((N_NODES, D_OUT), jnp.float32),
        jax.ShapeDtypeStruct((N_NODES, D_OUT), jnp.float32),
    ],
)


def _e1_body(ef_ref, we_ref, b1_ref, e1_ref):
    e1_ref[...] = lax.dot_general(ef_ref[...], we_ref[...], _DOT,
                                  preferred_element_type=jnp.float32) + b1_ref[...]


_BE = 8000
_e1_call = pl.pallas_call(
    _e1_body,
    grid=(N_EDGES // _BE,),
    in_specs=[
        pl.BlockSpec((_BE, D_EDGE), lambda i: (i, 0)),
        pl.BlockSpec((D_EDGE, D_OUT), lambda i: (0, 0)),
        pl.BlockSpec((1, D_OUT), lambda i: (0, 0)),
    ],
    out_specs=pl.BlockSpec((_BE, D_OUT), lambda i: (i, 0)),
    out_shape=jax.ShapeDtypeStruct((N_EDGES, D_OUT), jnp.float32),
)


def _sc_edge_body(p_hbm, q_hbm, e1_hbm, src_hbm, dst_hbm,
                  agg_out, cnt_out,
                  idx_s_a, idx_d_a, bp_a, bq_a, be_a,
                  idx_s_b, idx_d_b, bp_b, bq_b, be_b,
                  scd_a, scd_b, ones_v, zc_v,
                  agg_sh, cnt_sh,
                  sem_g_a, sem_i_a, sem_g_b, sem_i_b,
                  sem_sc_a, sem_scd_a, sem_sc_b, sem_scd_b):
    cid = lax.axis_index("c")
    sid = lax.axis_index("s")
    wid = sid * NC + cid
    ebase = wid * EPW

    set_a = (idx_s_a, idx_d_a, bp_a, bq_a, be_a, sem_g_a, sem_i_a,
             scd_a, sem_sc_a, sem_scd_a)
    set_b = (idx_s_b, idx_d_b, bp_b, bq_b, be_b, sem_g_b, sem_i_b,
             scd_b, sem_sc_b, sem_scd_b)

    zf = jnp.zeros((LANES,), jnp.float32)
    onef = jnp.ones((LANES,), jnp.float32)

    def _zfill(r, carry):
        for c in range(D_OUT // LANES):
            be_a[r, pl.ds(c * LANES, LANES)] = zf
        zc_v[r, pl.ds(0, LANES)] = zf
        ones_v[r, pl.ds(0, LANES)] = onef
        return carry

    lax.fori_loop(0, CHUNK, _zfill, 0)

    base_row = pl.multiple_of(sid * ROWS_PER_TILE, 8)
    for k in range(ROWS_PER_TILE // CHUNK):
        pltpu.sync_copy(be_a, agg_sh.at[pl.ds(base_row + k * CHUNK, CHUNK)])
        pltpu.sync_copy(zc_v, cnt_sh.at[pl.ds(base_row + k * CHUNK, CHUNK)])
    plsc.subcore_barrier()

    def _off(c):
        return pl.multiple_of(ebase + c * CHUNK, CHUNK)

    def issue_idx(c, s):
        idx_s, idx_d, sem_i = s[0], s[1], s[6]
        off = _off(c)
        pltpu.async_copy(src_hbm.at[pl.ds(off, CHUNK)], idx_s, sem_i)
        pltpu.async_copy(dst_hbm.at[pl.ds(off, CHUNK)], idx_d, sem_i)

    def wait_idx(s):
        idx_s, idx_d, sem_i = s[0], s[1], s[6]
        pltpu.make_async_copy(src_hbm.at[pl.ds(0, CHUNK)], idx_s, sem_i).wait()
        pltpu.make_async_copy(dst_hbm.at[pl.ds(0, CHUNK)], idx_d, sem_i).wait()

    def issue_gathers(c, s):
        idx_s, idx_d, bp, bq, be, sem_g = s[0], s[1], s[2], s[3], s[4], s[5]
        off = _off(c)
        pltpu.async_copy(e1_hbm.at[pl.ds(off, CHUNK)], be, sem_g)
        pltpu.async_copy(p_hbm.at[idx_s], bp, sem_g)
        pltpu.async_copy(q_hbm.at[idx_d], bq, sem_g)

    def wait_gathers(s):
        idx_s, idx_d, bp, bq, be, sem_g = s[0], s[1], s[2], s[3], s[4], s[5]
        pltpu.make_async_copy(e1_hbm.at[pl.ds(0, CHUNK)], be, sem_g).wait()
        pltpu.make_async_copy(p_hbm.at[idx_s], bp, sem_g).wait()
        pltpu.make_async_copy(q_hbm.at[idx_d], bq, sem_g).wait()

    def issue_scd(c, s):
        scd, sem_scd = s[7], s[9]
        pltpu.async_copy(dst_hbm.at[pl.ds(_off(c), CHUNK)], scd, sem_scd)

    def wait_scd(s):
        scd, sem_scd = s[7], s[9]
        pltpu.make_async_copy(dst_hbm.at[pl.ds(0, CHUNK)], scd, sem_scd).wait()

    def compute(s):
        bp, bq, be = s[2], s[3], s[4]

        @plsc.parallel_loop(0, CHUNK, 1, unroll=2)
        def _row(r):
            for c in range(D_OUT // LANES):
                sl = pl.ds(c * LANES, LANES)
                be[r, sl] = jnp.maximum(bp[r, sl] + bq[r, sl] + be[r, sl], 0.0)

    def issue_scatter(s):
        be, scd, sem_sc = s[4], s[7], s[8]
        pltpu.async_copy(be, agg_sh.at[scd], sem_sc, add=True)
        pltpu.async_copy(ones_v, cnt_sh.at[scd], sem_sc, add=True)

    def wait_scatter(s):
        be, scd, sem_sc = s[4], s[7], s[8]
        pltpu.make_async_copy(be, agg_sh.at[scd], sem_sc).wait()
        pltpu.make_async_copy(ones_v, cnt_sh.at[scd], sem_sc).wait()

    issue_idx(0, set_a)
    wait_idx(set_a)
    issue_gathers(0, set_a)
    issue_idx(1, set_b)

    def _pair(ci, carry):
        for k, (s, t) in ((0, (set_a, set_b)), (1, (set_b, set_a))):
            c = 2 * ci + k
            wait_gathers(s)

            @pl.when(c >= 1)
            def _():
                wait_scatter(t)

            @pl.when(c + 1 < NCHUNKS)
            def _():
                wait_idx(t)
                issue_gathers(c + 1, t)

            issue_scd(c, s)
            compute(s)
            wait_scd(s)
            issue_scatter(s)

            @pl.when(c + 2 < NCHUNKS)
            def _():
                issue_idx(c + 2, s)

        return carry

    lax.fori_loop(0, NPAIRS, _pair, 0)

    wait_scatter(set_b)
    plsc.subcore_barrier()

    pltpu.sync_copy(agg_sh.at[pl.ds(base_row, ROWS_PER_TILE)],
                    agg_out.at[cid, pl.ds(base_row, ROWS_PER_TILE)])
    pltpu.sync_copy(cnt_sh.at[pl.ds(base_row, ROWS_PER_TILE)],
                    cnt_out.at[cid, pl.ds(base_row, ROWS_PER_TILE)])


_sc_edge = pl.kernel(
    _sc_edge_body,
    mesh=plsc.VectorSubcoreMesh(core_axis_name="c", subcore_axis_name="s"),
    compiler_params=pltpu.CompilerParams(use_tc_tiling_on_sc=False,
                                         needs_layout_passes=False),
    out_type=[
        jax.ShapeDtypeStruct((NC, N_PAD, D_OUT), jnp.float32),
        jax.ShapeDtypeStruct((NC, N_PAD, CW), jnp.float32),
    ],
    scratch_types=[
        pltpu.VMEM((CHUNK,), jnp.int32),
        pltpu.VMEM((CHUNK,), jnp.int32),
        pltpu.VMEM((CHUNK, D_OUT), jnp.float32),
        pltpu.VMEM((CHUNK, D_OUT), jnp.float32),
        pltpu.VMEM((CHUNK, D_OUT), jnp.float32),
        pltpu.VMEM((CHUNK,), jnp.int32),
        pltpu.VMEM((CHUNK,), jnp.int32),
        pltpu.VMEM((CHUNK, D_OUT), jnp.float32),
        pltpu.VMEM((CHUNK, D_OUT), jnp.float32),
        pltpu.VMEM((CHUNK, D_OUT), jnp.float32),
        pltpu.VMEM((CHUNK,), jnp.int32),
        pltpu.VMEM((CHUNK,), jnp.int32),
        pltpu.VMEM((CHUNK, CW), jnp.float32),
        pltpu.VMEM((CHUNK, CW), jnp.float32),
        pltpu.VMEM_SHARED((N_PAD, D_OUT), jnp.float32),
        pltpu.VMEM_SHARED((N_PAD, CW), jnp.float32),
        pltpu.SemaphoreType.DMA,
        pltpu.SemaphoreType.DMA,
        pltpu.SemaphoreType.DMA,
        pltpu.SemaphoreType.DMA,
        pltpu.SemaphoreType.DMA,
        pltpu.SemaphoreType.DMA,
        pltpu.SemaphoreType.DMA,
        pltpu.SemaphoreType.DMA,
    ],
)


def _out_body(a0_ref, a1_ref, c0_ref, c1_ref, w2_ref, b2_ref, o_ref):
    agg = a0_ref[...] + a1_ref[...]
    deg = c0_ref[...][:, :1] + c1_ref[...][:, :1]
    o_ref[...] = lax.dot_general(agg, w2_ref[...], _DOT,
                                 preferred_element_type=jnp.float32,
                                 precision=lax.Precision.HIGHEST) + deg * b2_ref[...]


_BO = 1000
_out_call = pl.pallas_call(
    _out_body,
    grid=(N_NODES // _BO,),
    in_specs=[
        pl.BlockSpec((_BO, D_OUT), lambda i: (i, 0)),
        pl.BlockSpec((_BO, D_OUT), lambda i: (i, 0)),
        pl.BlockSpec((_BO, CW), lambda i: (i, 0)),
        pl.BlockSpec((_BO, CW), lambda i: (i, 0)),
        pl.BlockSpec((D_OUT, D_OUT), lambda i: (0, 0)),
        pl.BlockSpec((1, D_OUT), lambda i: (0, 0)),
    ],
    out_specs=pl.BlockSpec((_BO, D_OUT), lambda i: (i, 0)),
    out_shape=jax.ShapeDtypeStruct((N_NODES, D_OUT), jnp.float32),
)


def kernel(node_feats, edge_index, edge_feats, W1, b1, W2, b2):
    src = edge_index[0].astype(jnp.int32)
    dst = edge_index[1].astype(jnp.int32)
    p, q = _pq_call(node_feats, W1[:D_NODE], W1[D_NODE:2 * D_NODE])
    e1 = _e1_call(edge_feats, W1[2 * D_NODE:], b1.reshape(1, D_OUT))
    agg2, cnt2 = _sc_edge(p, q, e1, src, dst)
    out = _out_call(agg2[0], agg2[1], cnt2[0], cnt2[1],
                    W2, b2.reshape(1, D_OUT))
    return out


# speedup vs baseline: 21.4470x; 1.0465x over previous
import jax
import jax.numpy as jnp
from jax import lax
from jax.experimental import pallas as pl
from jax.experimental.pallas import tpu as pltpu
from jax.experimental.pallas import tpu_sc as plsc

N_NODES = 10000
N_EDGES = 320000
D_NODE = 128
D_EDGE = 16
D_OUT = 128

LANES = 16
CW = 16
NC = 2
NS = 16
NW = NC * NS
EPW = N_EDGES // NW
CHUNK = 40
NCHUNKS = EPW // CHUNK
NPAIRS = NCHUNKS // 2
N_PAD = 10240
ROWS_PER_TILE = N_PAD // NS

_DOT = (((1,), (0,)), ((), ()))


def _pq_body(x_ref, ws_ref, wd_ref, p_ref, q_ref):
    x = x_ref[...]
    p_ref[...] = lax.dot_general(x, ws_ref[...], _DOT,
                                 preferred_element_type=jnp.float32)
    q_ref[...] = lax.dot_general(x, wd_ref[...], _DOT,
                                 preferred_element_type=jnp.float32)


_BN = 2000
_pq_call = pl.pallas_call(
    _pq_body,
    grid=(N_NODES // _BN,),
    in_specs=[
        pl.BlockSpec((_BN, D_NODE), lambda i: (i, 0)),
        pl.BlockSpec((D_NODE, D_OUT), lambda i: (0, 0)),
        pl.BlockSpec((D_NODE, D_OUT), lambda i: (0, 0)),
    ],
    out_specs=[
        pl.BlockSpec((_BN, D_OUT), lambda i: (i, 0)),
        pl.BlockSpec((_BN, D_OUT), lambda i: (i, 0)),
    ],
    out_shape=[
        jax.ShapeDtypeStruct((N_NODES, D_OUT), jnp.float32),
        jax.ShapeDtypeStruct((N_NODES, D_OUT), jnp.float32),
    ],
)


def _pack16(y):
    return lax.convert_element_type(
        lax.bitcast_convert_type(y.astype(jnp.bfloat16), jnp.uint16),
        jnp.uint32)


def _e1_body(ef_ref, we_ref, b1_ref, e1_ref):
    ef = ef_ref[...]
    b1 = b1_ref[...]
    ea = lax.dot_general(ef[:, :D_EDGE], we_ref[...], _DOT,
                         preferred_element_type=jnp.float32) + b1
    eb = lax.dot_general(ef[:, D_EDGE:], we_ref[...], _DOT,
                         preferred_element_type=jnp.float32) + b1
    word = _pack16(ea) | (_pack16(eb) << 16)
    e1_ref[...] = lax.bitcast_convert_type(word, jnp.int32)


_BE = 8000
_e1_call = pl.pallas_call(
    _e1_body,
    grid=(N_EDGES // 2 // _BE,),
    in_specs=[
        pl.BlockSpec((_BE, 2 * D_EDGE), lambda i: (i, 0)),
        pl.BlockSpec((D_EDGE, D_OUT), lambda i: (0, 0)),
        pl.BlockSpec((1, D_OUT), lambda i: (0, 0)),
    ],
    out_specs=pl.BlockSpec((_BE, D_OUT), lambda i: (i, 0)),
    out_shape=jax.ShapeDtypeStruct((N_EDGES // 2, D_OUT), jnp.int32),
)


def _sc_edge_body(p_hbm, q_hbm, e1_hbm, src_hbm, dst_hbm,
                  agg_out, cnt_out,
                  idx_s_a, idx_d_a, bp_a, bq_a, be_a, out_a,
                  idx_s_b, idx_d_b, bp_b, bq_b, be_b, out_b,
                  scd_a, scd_b, ones_v, zc_v,
                  agg_sh, cnt_sh,
                  sem_g_a, sem_i_a, sem_g_b, sem_i_b,
                  sem_sc_a, sem_scd_a, sem_sc_b, sem_scd_b):
    cid = lax.axis_index("c")
    sid = lax.axis_index("s")
    wid = sid * NC + cid
    ebase = wid * EPW
    ebase2 = wid * (EPW // 2)

    set_a = (idx_s_a, idx_d_a, bp_a, bq_a, be_a, sem_g_a, sem_i_a,
             scd_a, sem_sc_a, sem_scd_a, out_a)
    set_b = (idx_s_b, idx_d_b, bp_b, bq_b, be_b, sem_g_b, sem_i_b,
             scd_b, sem_sc_b, sem_scd_b, out_b)

    zf = jnp.zeros((LANES,), jnp.float32)
    onef = jnp.ones((LANES,), jnp.float32)

    def _zfill(r, carry):
        for c in range(D_OUT // LANES):
            out_a[r, pl.ds(c * LANES, LANES)] = zf
        zc_v[r, pl.ds(0, LANES)] = zf
        ones_v[r, pl.ds(0, LANES)] = onef
        return carry

    lax.fori_loop(0, CHUNK, _zfill, 0)

    base_row = pl.multiple_of(sid * ROWS_PER_TILE, 8)
    for k in range(ROWS_PER_TILE // CHUNK):
        pltpu.sync_copy(out_a, agg_sh.at[pl.ds(base_row + k * CHUNK, CHUNK)])
        pltpu.sync_copy(zc_v, cnt_sh.at[pl.ds(base_row + k * CHUNK, CHUNK)])
    plsc.subcore_barrier()

    def _off(c):
        return pl.multiple_of(ebase + c * CHUNK, CHUNK)

    def issue_idx(c, s):
        idx_s, idx_d, sem_i = s[0], s[1], s[6]
        off = _off(c)
        pltpu.async_copy(src_hbm.at[pl.ds(off, CHUNK)], idx_s, sem_i)
        pltpu.async_copy(dst_hbm.at[pl.ds(off, CHUNK)], idx_d, sem_i)

    def wait_idx(s):
        idx_s, idx_d, sem_i = s[0], s[1], s[6]
        pltpu.make_async_copy(src_hbm.at[pl.ds(0, CHUNK)], idx_s, sem_i).wait()
        pltpu.make_async_copy(dst_hbm.at[pl.ds(0, CHUNK)], idx_d, sem_i).wait()

    def issue_gathers(c, s):
        idx_s, idx_d, bp, bq, be, sem_g = s[0], s[1], s[2], s[3], s[4], s[5]
        off = _off(c)
        off2 = pl.multiple_of(ebase2 + c * (CHUNK // 2), CHUNK // 2)
        pltpu.async_copy(e1_hbm.at[pl.ds(off2, CHUNK // 2)], be, sem_g)
        pltpu.async_copy(p_hbm.at[idx_s], bp, sem_g)
        pltpu.async_copy(q_hbm.at[idx_d], bq, sem_g)

    def wait_gathers(s):
        idx_s, idx_d, bp, bq, be, sem_g = s[0], s[1], s[2], s[3], s[4], s[5]
        pltpu.make_async_copy(e1_hbm.at[pl.ds(0, CHUNK // 2)], be, sem_g).wait()
        pltpu.make_async_copy(p_hbm.at[idx_s], bp, sem_g).wait()
        pltpu.make_async_copy(q_hbm.at[idx_d], bq, sem_g).wait()

    def issue_scd(c, s):
        scd, sem_scd = s[7], s[9]
        pltpu.async_copy(dst_hbm.at[pl.ds(_off(c), CHUNK)], scd, sem_scd)

    def wait_scd(s):
        scd, sem_scd = s[7], s[9]
        pltpu.make_async_copy(dst_hbm.at[pl.ds(0, CHUNK)], scd, sem_scd).wait()

    def compute(s):
        bp, bq, be, out = s[2], s[3], s[4], s[10]

        @plsc.parallel_loop(0, CHUNK // 2, 1, unroll=2)
        def _pairrow(pr):
            r0 = 2 * pr
            r1 = r0 + 1
            for c in range(D_OUT // LANES):
                sl = pl.ds(c * LANES, LANES)
                ea, eb = plsc.unpack(plsc.bitcast(be[pr, sl], jnp.bfloat16),
                                     format=plsc.PackFormat.INTERLEAVED)
                out[r0, sl] = jnp.maximum(bp[r0, sl] + bq[r0, sl] + ea, 0.0)
                out[r1, sl] = jnp.maximum(bp[r1, sl] + bq[r1, sl] + eb, 0.0)

    def issue_scatter(s):
        scd, sem_sc, out = s[7], s[8], s[10]
        pltpu.async_copy(out, agg_sh.at[scd], sem_sc, add=True)
        pltpu.async_copy(ones_v, cnt_sh.at[scd], sem_sc, add=True)

    def wait_scatter(s):
        scd, sem_sc, out = s[7], s[8], s[10]
        pltpu.make_async_copy(out, agg_sh.at[scd], sem_sc).wait()
        pltpu.make_async_copy(ones_v, cnt_sh.at[scd], sem_sc).wait()

    issue_idx(0, set_a)
    wait_idx(set_a)
    issue_gathers(0, set_a)
    issue_idx(1, set_b)

    def _pair(ci, carry):
        for k, (s, t) in ((0, (set_a, set_b)), (1, (set_b, set_a))):
            c = 2 * ci + k
            wait_gathers(s)

            @pl.when(c >= 1)
            def _():
                wait_scatter(t)

            @pl.when(c + 1 < NCHUNKS)
            def _():
                wait_idx(t)
                issue_gathers(c + 1, t)

            issue_scd(c, s)
            compute(s)
            wait_scd(s)
            issue_scatter(s)

            @pl.when(c + 2 < NCHUNKS)
            def _():
                issue_idx(c + 2, s)

        return carry

    lax.fori_loop(0, NPAIRS, _pair, 0)

    wait_scatter(set_b)
    plsc.subcore_barrier()

    pltpu.sync_copy(agg_sh.at[pl.ds(base_row, ROWS_PER_TILE)],
                    agg_out.at[cid, pl.ds(base_row, ROWS_PER_TILE)])
    pltpu.sync_copy(cnt_sh.at[pl.ds(base_row, ROWS_PER_TILE)],
                    cnt_out.at[cid, pl.ds(base_row, ROWS_PER_TILE)])


_sc_edge = pl.kernel(
    _sc_edge_body,
    mesh=plsc.VectorSubcoreMesh(core_axis_name="c", subcore_axis_name="s"),
    compiler_params=pltpu.CompilerParams(use_tc_tiling_on_sc=False,
                                         needs_layout_passes=False),
    out_type=[
        jax.ShapeDtypeStruct((NC, N_PAD, D_OUT), jnp.float32),
        jax.ShapeDtypeStruct((NC, N_PAD, CW), jnp.float32),
    ],
    scratch_types=[
        pltpu.VMEM((CHUNK,), jnp.int32),
        pltpu.VMEM((CHUNK,), jnp.int32),
        pltpu.VMEM((CHUNK, D_OUT), jnp.float32),
        pltpu.VMEM((CHUNK, D_OUT), jnp.float32),
        pltpu.VMEM((CHUNK // 2, D_OUT), jnp.int32),
        pltpu.VMEM((CHUNK, D_OUT), jnp.float32),
        pltpu.VMEM((CHUNK,), jnp.int32),
        pltpu.VMEM((CHUNK,), jnp.int32),
        pltpu.VMEM((CHUNK, D_OUT), jnp.float32),
        pltpu.VMEM((CHUNK, D_OUT), jnp.float32),
        pltpu.VMEM((CHUNK // 2, D_OUT), jnp.int32),
        pltpu.VMEM((CHUNK, D_OUT), jnp.float32),
        pltpu.VMEM((CHUNK,), jnp.int32),
        pltpu.VMEM((CHUNK,), jnp.int32),
        pltpu.VMEM((CHUNK, CW), jnp.float32),
        pltpu.VMEM((CHUNK, CW), jnp.float32),
        pltpu.VMEM_SHARED((N_PAD, D_OUT), jnp.float32),
        pltpu.VMEM_SHARED((N_PAD, CW), jnp.float32),
        pltpu.SemaphoreType.DMA,
        pltpu.SemaphoreType.DMA,
        pltpu.SemaphoreType.DMA,
        pltpu.SemaphoreType.DMA,
        pltpu.SemaphoreType.DMA,
        pltpu.SemaphoreType.DMA,
        pltpu.SemaphoreType.DMA,
        pltpu.SemaphoreType.DMA,
    ],
)


def _out_body(a0_ref, a1_ref, c0_ref, c1_ref, w2_ref, b2_ref, o_ref):
    agg = a0_ref[...] + a1_ref[...]
    deg = c0_ref[...][:, :1] + c1_ref[...][:, :1]
    o_ref[...] = lax.dot_general(agg, w2_ref[...], _DOT,
                                 preferred_element_type=jnp.float32,
                                 precision=lax.Precision.HIGHEST) + deg * b2_ref[...]


_BO = 1000
_out_call = pl.pallas_call(
    _out_body,
    grid=(N_NODES // _BO,),
    in_specs=[
        pl.BlockSpec((_BO, D_OUT), lambda i: (i, 0)),
        pl.BlockSpec((_BO, D_OUT), lambda i: (i, 0)),
        pl.BlockSpec((_BO, CW), lambda i: (i, 0)),
        pl.BlockSpec((_BO, CW), lambda i: (i, 0)),
        pl.BlockSpec((D_OUT, D_OUT), lambda i: (0, 0)),
        pl.BlockSpec((1, D_OUT), lambda i: (0, 0)),
    ],
    out_specs=pl.BlockSpec((_BO, D_OUT), lambda i: (i, 0)),
    out_shape=jax.ShapeDtypeStruct((N_NODES, D_OUT), jnp.float32),
)


def kernel(node_feats, edge_index, edge_feats, W1, b1, W2, b2):
    src = edge_index[0].astype(jnp.int32)
    dst = edge_index[1].astype(jnp.int32)
    p, q = _pq_call(node_feats, W1[:D_NODE], W1[D_NODE:2 * D_NODE])
    e1 = _e1_call(edge_feats.reshape(N_EDGES // 2, 2 * D_EDGE),
                  W1[2 * D_NODE:], b1.reshape(1, D_OUT))
    agg2, cnt2 = _sc_edge(p, q, e1, src, dst)
    out = _out_call(agg2[0], agg2[1], cnt2[0], cnt2[1],
                    W2, b2.reshape(1, D_OUT))
    return out


# speedup vs baseline: 22.3428x; 1.0418x over previous
import jax
import jax.numpy as jnp
from jax import lax
from jax.experimental import pallas as pl
from jax.experimental.pallas import tpu as pltpu
from jax.experimental.pallas import tpu_sc as plsc

N_NODES = 10000
N_EDGES = 320000
D_NODE = 128
D_EDGE = 16
D_OUT = 128

LANES = 16
CW = 16
NC = 2
NS = 16
NW = NC * NS
EPW = N_EDGES // NW
CHUNK = 40
NCHUNKS = EPW // CHUNK
NPAIRS = NCHUNKS // 2
N_PAD = 10240
ROWS_PER_TILE = N_PAD // NS

_DOT = (((1,), (0,)), ((), ()))


def _pq_body(x_ref, ws_ref, wd_ref, p_ref, q_ref):
    x = x_ref[...]
    p_ref[...] = lax.dot_general(x, ws_ref[...], _DOT,
                                 preferred_element_type=jnp.float32)
    q_ref[...] = lax.dot_general(x, wd_ref[...], _DOT,
                                 preferred_element_type=jnp.float32)


_BN = 2000
_pq_call = pl.pallas_call(
    _pq_body,
    grid=(N_NODES // _BN,),
    in_specs=[
        pl.BlockSpec((_BN, D_NODE), lambda i: (i, 0)),
        pl.BlockSpec((D_NODE, D_OUT), lambda i: (0, 0)),
        pl.BlockSpec((D_NODE, D_OUT), lambda i: (0, 0)),
    ],
    out_specs=[
        pl.BlockSpec((_BN, D_OUT), lambda i: (i, 0)),
        pl.BlockSpec((_BN, D_OUT), lambda i: (i, 0)),
    ],
    out_shape=[
        jax.ShapeDtypeStruct((N_NODES, D_OUT), jnp.float32),
        jax.ShapeDtypeStruct((N_NODES, D_OUT), jnp.float32),
    ],
)


def _pack16(y):
    return lax.convert_element_type(
        lax.bitcast_convert_type(y.astype(jnp.bfloat16), jnp.uint16),
        jnp.uint32)


def _e1_body(ef_ref, we_ref, b1_ref, e1_ref):
    ef = ef_ref[...]
    b1 = b1_ref[...]
    ea = lax.dot_general(ef[:, :D_EDGE], we_ref[...], _DOT,
                         preferred_element_type=jnp.float32) + b1
    eb = lax.dot_general(ef[:, D_EDGE:], we_ref[...], _DOT,
                         preferred_element_type=jnp.float32) + b1
    word = _pack16(ea) | (_pack16(eb) << 16)
    e1_ref[...] = lax.bitcast_convert_type(word, jnp.int32)


_BE = 8000
_e1_call = pl.pallas_call(
    _e1_body,
    grid=(N_EDGES // 2 // _BE,),
    in_specs=[
        pl.BlockSpec((_BE, 2 * D_EDGE), lambda i: (i, 0)),
        pl.BlockSpec((D_EDGE, D_OUT), lambda i: (0, 0)),
        pl.BlockSpec((1, D_OUT), lambda i: (0, 0)),
    ],
    out_specs=pl.BlockSpec((_BE, D_OUT), lambda i: (i, 0)),
    out_shape=jax.ShapeDtypeStruct((N_EDGES // 2, D_OUT), jnp.int32),
)


def _sc_edge_body(p_hbm, q_hbm, e1_hbm, src_hbm, dst_hbm,
                  agg_out, cnt_out,
                  idx_s_a, idx_d_a, bp_a, bq_a, be_a, out_a,
                  idx_s_b, idx_d_b, bp_b, bq_b, be_b, out_b,
                  scd_a, scd_b, ones_v, zc_v,
                  agg_sh, cnt_sh,
                  sem_g_a, sem_i_a, sem_g_b, sem_i_b,
                  sem_sc_a, sem_scd_a, sem_sc_b, sem_scd_b):
    cid = lax.axis_index("c")
    sid = lax.axis_index("s")
    wid = sid * NC + cid
    ebase = wid * EPW
    ebase2 = wid * (EPW // 2)

    set_a = (idx_s_a, idx_d_a, bp_a, bq_a, be_a, sem_g_a, sem_i_a,
             scd_a, sem_sc_a, sem_scd_a, out_a)
    set_b = (idx_s_b, idx_d_b, bp_b, bq_b, be_b, sem_g_b, sem_i_b,
             scd_b, sem_sc_b, sem_scd_b, out_b)

    zf = jnp.zeros((LANES,), jnp.float32)
    onef = jnp.ones((LANES,), jnp.float32)

    def _zfill(r, carry):
        for c in range(D_OUT // LANES):
            out_a[r, pl.ds(c * LANES, LANES)] = zf
        zc_v[r, pl.ds(0, LANES)] = zf
        ones_v[r, pl.ds(0, LANES)] = onef
        return carry

    lax.fori_loop(0, CHUNK, _zfill, 0)

    base_row = pl.multiple_of(sid * ROWS_PER_TILE, 8)
    for k in range(ROWS_PER_TILE // CHUNK):
        pltpu.sync_copy(out_a, agg_sh.at[pl.ds(base_row + k * CHUNK, CHUNK)])
        pltpu.sync_copy(zc_v, cnt_sh.at[pl.ds(base_row + k * CHUNK, CHUNK)])
    plsc.subcore_barrier()

    def _off(c):
        return pl.multiple_of(ebase + c * CHUNK, CHUNK)

    def issue_idx(c, s):
        idx_s, idx_d, sem_i = s[0], s[1], s[6]
        off = _off(c)
        pltpu.async_copy(src_hbm.at[pl.ds(off, CHUNK)], idx_s, sem_i)
        pltpu.async_copy(dst_hbm.at[pl.ds(off, CHUNK)], idx_d, sem_i)

    def wait_idx(s):
        idx_s, idx_d, sem_i = s[0], s[1], s[6]
        pltpu.make_async_copy(src_hbm.at[pl.ds(0, CHUNK)], idx_s, sem_i).wait()
        pltpu.make_async_copy(dst_hbm.at[pl.ds(0, CHUNK)], idx_d, sem_i).wait()

    def issue_gathers(c, s):
        idx_s, idx_d, bp, bq, be, sem_g = s[0], s[1], s[2], s[3], s[4], s[5]
        off = _off(c)
        off2 = pl.multiple_of(ebase2 + c * (CHUNK // 2), CHUNK // 2)
        pltpu.async_copy(e1_hbm.at[pl.ds(off2, CHUNK // 2)], be, sem_g)
        pltpu.async_copy(p_hbm.at[idx_s], bp, sem_g)
        pltpu.async_copy(q_hbm.at[idx_d], bq, sem_g)

    def wait_gathers(s):
        idx_s, idx_d, bp, bq, be, sem_g = s[0], s[1], s[2], s[3], s[4], s[5]
        pltpu.make_async_copy(e1_hbm.at[pl.ds(0, CHUNK // 2)], be, sem_g).wait()
        pltpu.make_async_copy(p_hbm.at[idx_s], bp, sem_g).wait()
        pltpu.make_async_copy(q_hbm.at[idx_d], bq, sem_g).wait()

    def issue_scd(c, s):
        scd, sem_scd = s[7], s[9]
        pltpu.async_copy(dst_hbm.at[pl.ds(_off(c), CHUNK)], scd, sem_scd)

    def wait_scd(s):
        scd, sem_scd = s[7], s[9]
        pltpu.make_async_copy(dst_hbm.at[pl.ds(0, CHUNK)], scd, sem_scd).wait()

    def compute(s):
        bp, bq, be, out = s[2], s[3], s[4], s[10]

        @plsc.parallel_loop(0, CHUNK // 2, 1, unroll=2)
        def _pairrow(pr):
            r0 = 2 * pr
            r1 = r0 + 1
            for c in range(D_OUT // LANES):
                sl = pl.ds(c * LANES, LANES)
                ea, eb = plsc.unpack(plsc.bitcast(be[pr, sl], jnp.bfloat16),
                                     format=plsc.PackFormat.INTERLEAVED)
                out[r0, sl] = jnp.maximum(bp[r0, sl] + bq[r0, sl] + ea, 0.0)
                out[r1, sl] = jnp.maximum(bp[r1, sl] + bq[r1, sl] + eb, 0.0)

    def issue_scatter(s):
        scd, sem_sc, out = s[7], s[8], s[10]
        pltpu.async_copy(out, agg_sh.at[scd], sem_sc, add=True)
        pltpu.async_copy(ones_v, cnt_sh.at[scd], sem_sc, add=True)

    def wait_scatter(s):
        scd, sem_sc, out = s[7], s[8], s[10]
        pltpu.make_async_copy(out, agg_sh.at[scd], sem_sc).wait()
        pltpu.make_async_copy(ones_v, cnt_sh.at[scd], sem_sc).wait()

    issue_idx(0, set_a)
    wait_idx(set_a)
    issue_gathers(0, set_a)
    issue_idx(1, set_b)

    def _pair(ci, carry):
        for k, (s, t) in ((0, (set_a, set_b)), (1, (set_b, set_a))):
            c = 2 * ci + k
            issue_scd(c, s)

            @pl.when(c + 1 < NCHUNKS)
            def _():
                wait_idx(t)
                issue_gathers(c + 1, t)

            wait_gathers(s)

            @pl.when(c >= 1)
            def _():
                wait_scatter(t)

            compute(s)
            wait_scd(s)
            issue_scatter(s)

            @pl.when(c + 2 < NCHUNKS)
            def _():
                issue_idx(c + 2, s)

        return carry

    lax.fori_loop(0, NPAIRS, _pair, 0)

    wait_scatter(set_b)
    plsc.subcore_barrier()

    pltpu.sync_copy(agg_sh.at[pl.ds(base_row, ROWS_PER_TILE)],
                    agg_out.at[cid, pl.ds(base_row, ROWS_PER_TILE)])
    pltpu.sync_copy(cnt_sh.at[pl.ds(base_row, ROWS_PER_TILE)],
                    cnt_out.at[cid, pl.ds(base_row, ROWS_PER_TILE)])


_sc_edge = pl.kernel(
    _sc_edge_body,
    mesh=plsc.VectorSubcoreMesh(core_axis_name="c", subcore_axis_name="s"),
    compiler_params=pltpu.CompilerParams(use_tc_tiling_on_sc=False,
                                         needs_layout_passes=False),
    out_type=[
        jax.ShapeDtypeStruct((NC, N_PAD, D_OUT), jnp.float32),
        jax.ShapeDtypeStruct((NC, N_PAD, CW), jnp.float32),
    ],
    scratch_types=[
        pltpu.VMEM((CHUNK,), jnp.int32),
        pltpu.VMEM((CHUNK,), jnp.int32),
        pltpu.VMEM((CHUNK, D_OUT), jnp.float32),
        pltpu.VMEM((CHUNK, D_OUT), jnp.float32),
        pltpu.VMEM((CHUNK // 2, D_OUT), jnp.int32),
        pltpu.VMEM((CHUNK, D_OUT), jnp.float32),
        pltpu.VMEM((CHUNK,), jnp.int32),
        pltpu.VMEM((CHUNK,), jnp.int32),
        pltpu.VMEM((CHUNK, D_OUT), jnp.float32),
        pltpu.VMEM((CHUNK, D_OUT), jnp.float32),
        pltpu.VMEM((CHUNK // 2, D_OUT), jnp.int32),
        pltpu.VMEM((CHUNK, D_OUT), jnp.float32),
        pltpu.VMEM((CHUNK,), jnp.int32),
        pltpu.VMEM((CHUNK,), jnp.int32),
        pltpu.VMEM((CHUNK, CW), jnp.float32),
        pltpu.VMEM((CHUNK, CW), jnp.float32),
        pltpu.VMEM_SHARED((N_PAD, D_OUT), jnp.float32),
        pltpu.VMEM_SHARED((N_PAD, CW), jnp.float32),
        pltpu.SemaphoreType.DMA,
        pltpu.SemaphoreType.DMA,
        pltpu.SemaphoreType.DMA,
        pltpu.SemaphoreType.DMA,
        pltpu.SemaphoreType.DMA,
        pltpu.SemaphoreType.DMA,
        pltpu.SemaphoreType.DMA,
        pltpu.SemaphoreType.DMA,
    ],
)


def _out_body(a0_ref, a1_ref, c0_ref, c1_ref, w2_ref, b2_ref, o_ref):
    agg = a0_ref[...] + a1_ref[...]
    deg = c0_ref[...][:, :1] + c1_ref[...][:, :1]
    o_ref[...] = lax.dot_general(agg, w2_ref[...], _DOT,
                                 preferred_element_type=jnp.float32,
                                 precision=lax.Precision.HIGHEST) + deg * b2_ref[...]


_BO = 1000
_out_call = pl.pallas_call(
    _out_body,
    grid=(N_NODES // _BO,),
    in_specs=[
        pl.BlockSpec((_BO, D_OUT), lambda i: (i, 0)),
        pl.BlockSpec((_BO, D_OUT), lambda i: (i, 0)),
        pl.BlockSpec((_BO, CW), lambda i: (i, 0)),
        pl.BlockSpec((_BO, CW), lambda i: (i, 0)),
        pl.BlockSpec((D_OUT, D_OUT), lambda i: (0, 0)),
        pl.BlockSpec((1, D_OUT), lambda i: (0, 0)),
    ],
    out_specs=pl.BlockSpec((_BO, D_OUT), lambda i: (i, 0)),
    out_shape=jax.ShapeDtypeStruct((N_NODES, D_OUT), jnp.float32),
)


def kernel(node_feats, edge_index, edge_feats, W1, b1, W2, b2):
    src = edge_index[0].astype(jnp.int32)
    dst = edge_index[1].astype(jnp.int32)
    p, q = _pq_call(node_feats, W1[:D_NODE], W1[D_NODE:2 * D_NODE])
    e1 = _e1_call(edge_feats.reshape(N_EDGES // 2, 2 * D_EDGE),
                  W1[2 * D_NODE:], b1.reshape(1, D_OUT))
    agg2, cnt2 = _sc_edge(p, q, e1, src, dst)
    out = _out_call(agg2[0], agg2[1], cnt2[0], cnt2[1],
                    W2, b2.reshape(1, D_OUT))
    return out


# speedup vs baseline: 22.4260x; 1.0037x over previous
import jax
import jax.numpy as jnp
from jax import lax
from jax.experimental import pallas as pl
from jax.experimental.pallas import tpu as pltpu
from jax.experimental.pallas import tpu_sc as plsc

N_NODES = 10000
N_EDGES = 320000
D_NODE = 128
D_EDGE = 16
D_OUT = 128

LANES = 16
CW = 16
NC = 2
NS = 16
NW = NC * NS
EPW = N_EDGES // NW
CHUNK = 40
NCHUNKS = EPW // CHUNK
NPAIRS = NCHUNKS // 2
N_PAD = 10240
ROWS_PER_TILE = N_PAD // NS

_DOT = (((1,), (0,)), ((), ()))


def _pq_body(x_ref, ws_ref, wd_ref, p_ref, q_ref):
    x = x_ref[...]
    p_ref[...] = lax.dot_general(x, ws_ref[...], _DOT,
                                 preferred_element_type=jnp.float32)
    q_ref[...] = lax.dot_general(x, wd_ref[...], _DOT,
                                 preferred_element_type=jnp.float32)


_BN = 2000
_pq_call = pl.pallas_call(
    _pq_body,
    grid=(N_NODES // _BN,),
    in_specs=[
        pl.BlockSpec((_BN, D_NODE), lambda i: (i, 0)),
        pl.BlockSpec((D_NODE, D_OUT), lambda i: (0, 0)),
        pl.BlockSpec((D_NODE, D_OUT), lambda i: (0, 0)),
    ],
    out_specs=[
        pl.BlockSpec((_BN, D_OUT), lambda i: (i, 0)),
        pl.BlockSpec((_BN, D_OUT), lambda i: (i, 0)),
    ],
    out_shape=[
        jax.ShapeDtypeStruct((N_NODES, D_OUT), jnp.float32),
        jax.ShapeDtypeStruct((N_NODES, D_OUT), jnp.float32),
    ],
)


def _pack16(y):
    return lax.convert_element_type(
        lax.bitcast_convert_type(y.astype(jnp.bfloat16), jnp.uint16),
        jnp.uint32)


def _e1_body(ef_ref, we_ref, b1_ref, e1_ref):
    ef = ef_ref[...]
    b1 = b1_ref[...]
    ea = lax.dot_general(ef[:, :D_EDGE], we_ref[...], _DOT,
                         preferred_element_type=jnp.float32) + b1
    eb = lax.dot_general(ef[:, D_EDGE:], we_ref[...], _DOT,
                         preferred_element_type=jnp.float32) + b1
    word = _pack16(ea) | (_pack16(eb) << 16)
    e1_ref[...] = lax.bitcast_convert_type(word, jnp.int32)


_BE = 8000
_e1_call = pl.pallas_call(
    _e1_body,
    grid=(N_EDGES // 2 // _BE,),
    in_specs=[
        pl.BlockSpec((_BE, 2 * D_EDGE), lambda i: (i, 0)),
        pl.BlockSpec((D_EDGE, D_OUT), lambda i: (0, 0)),
        pl.BlockSpec((1, D_OUT), lambda i: (0, 0)),
    ],
    out_specs=pl.BlockSpec((_BE, D_OUT), lambda i: (i, 0)),
    out_shape=jax.ShapeDtypeStruct((N_EDGES // 2, D_OUT), jnp.int32),
)


def _sc_edge_body(p_hbm, q_hbm, e1_hbm, src_hbm, dst_hbm,
                  agg_out, cnt_out,
                  idx_s_a, idx_d_a, bp_a, bq_a, be_a, out_a,
                  idx_s_b, idx_d_b, bp_b, bq_b, be_b, out_b,
                  scd_a, scd_b, ones_v, zc_v,
                  agg_sh, cnt_sh,
                  sem_g_a, sem_i_a, sem_g_b, sem_i_b,
                  sem_sc_a, sem_scd_a, sem_sc_b, sem_scd_b):
    cid = lax.axis_index("c")
    sid = lax.axis_index("s")
    wid = sid * NC + cid
    ebase = wid * EPW
    ebase2 = wid * (EPW // 2)

    set_a = (idx_s_a, idx_d_a, bp_a, bq_a, be_a, sem_g_a, sem_i_a,
             scd_a, sem_sc_a, sem_scd_a, out_a)
    set_b = (idx_s_b, idx_d_b, bp_b, bq_b, be_b, sem_g_b, sem_i_b,
             scd_b, sem_sc_b, sem_scd_b, out_b)

    zf = jnp.zeros((LANES,), jnp.float32)
    onef = jnp.ones((LANES,), jnp.float32)

    def _zfill(r, carry):
        for c in range(D_OUT // LANES):
            out_a[r, pl.ds(c * LANES, LANES)] = zf
        zc_v[r, pl.ds(0, LANES)] = zf
        ones_v[r, pl.ds(0, LANES)] = onef
        return carry

    lax.fori_loop(0, CHUNK, _zfill, 0)

    base_row = pl.multiple_of(sid * ROWS_PER_TILE, 8)
    for k in range(ROWS_PER_TILE // CHUNK):
        pltpu.async_copy(out_a, agg_sh.at[pl.ds(base_row + k * CHUNK, CHUNK)],
                         sem_g_a)
        pltpu.async_copy(zc_v, cnt_sh.at[pl.ds(base_row + k * CHUNK, CHUNK)],
                         sem_i_a)
    for k in range(ROWS_PER_TILE // CHUNK):
        pltpu.make_async_copy(
            out_a, agg_sh.at[pl.ds(base_row + k * CHUNK, CHUNK)],
            sem_g_a).wait()
        pltpu.make_async_copy(
            zc_v, cnt_sh.at[pl.ds(base_row + k * CHUNK, CHUNK)],
            sem_i_a).wait()
    plsc.subcore_barrier()

    def _off(c):
        return pl.multiple_of(ebase + c * CHUNK, CHUNK)

    def issue_idx(c, s):
        idx_s, idx_d, sem_i = s[0], s[1], s[6]
        off = _off(c)
        pltpu.async_copy(src_hbm.at[pl.ds(off, CHUNK)], idx_s, sem_i)
        pltpu.async_copy(dst_hbm.at[pl.ds(off, CHUNK)], idx_d, sem_i)

    def wait_idx(s):
        idx_s, idx_d, sem_i = s[0], s[1], s[6]
        pltpu.make_async_copy(src_hbm.at[pl.ds(0, CHUNK)], idx_s, sem_i).wait()
        pltpu.make_async_copy(dst_hbm.at[pl.ds(0, CHUNK)], idx_d, sem_i).wait()

    def issue_gathers(c, s):
        idx_s, idx_d, bp, bq, be, sem_g = s[0], s[1], s[2], s[3], s[4], s[5]
        off = _off(c)
        off2 = pl.multiple_of(ebase2 + c * (CHUNK // 2), CHUNK // 2)
        pltpu.async_copy(e1_hbm.at[pl.ds(off2, CHUNK // 2)], be, sem_g)
        pltpu.async_copy(p_hbm.at[idx_s], bp, sem_g)
        pltpu.async_copy(q_hbm.at[idx_d], bq, sem_g)

    def wait_gathers(s):
        idx_s, idx_d, bp, bq, be, sem_g = s[0], s[1], s[2], s[3], s[4], s[5]
        pltpu.make_async_copy(e1_hbm.at[pl.ds(0, CHUNK // 2)], be, sem_g).wait()
        pltpu.make_async_copy(p_hbm.at[idx_s], bp, sem_g).wait()
        pltpu.make_async_copy(q_hbm.at[idx_d], bq, sem_g).wait()

    def issue_scd(c, s):
        scd, sem_scd = s[7], s[9]
        pltpu.async_copy(dst_hbm.at[pl.ds(_off(c), CHUNK)], scd, sem_scd)

    def wait_scd(s):
        scd, sem_scd = s[7], s[9]
        pltpu.make_async_copy(dst_hbm.at[pl.ds(0, CHUNK)], scd, sem_scd).wait()

    def compute(s):
        bp, bq, be, out = s[2], s[3], s[4], s[10]

        @plsc.parallel_loop(0, CHUNK // 2, 1, unroll=2)
        def _pairrow(pr):
            r0 = 2 * pr
            r1 = r0 + 1
            for c in range(D_OUT // LANES):
                sl = pl.ds(c * LANES, LANES)
                ea, eb = plsc.unpack(plsc.bitcast(be[pr, sl], jnp.bfloat16),
                                     format=plsc.PackFormat.INTERLEAVED)
                out[r0, sl] = jnp.maximum(bp[r0, sl] + bq[r0, sl] + ea, 0.0)
                out[r1, sl] = jnp.maximum(bp[r1, sl] + bq[r1, sl] + eb, 0.0)

    def issue_scatter(s):
        scd, sem_sc, out = s[7], s[8], s[10]
        pltpu.async_copy(out, agg_sh.at[scd], sem_sc, add=True)
        pltpu.async_copy(ones_v, cnt_sh.at[scd], sem_sc, add=True)

    def wait_scatter(s):
        scd, sem_sc, out = s[7], s[8], s[10]
        pltpu.make_async_copy(out, agg_sh.at[scd], sem_sc).wait()
        pltpu.make_async_copy(ones_v, cnt_sh.at[scd], sem_sc).wait()

    issue_idx(0, set_a)
    wait_idx(set_a)
    issue_gathers(0, set_a)
    issue_idx(1, set_b)

    def _pair(ci, carry):
        for k, (s, t) in ((0, (set_a, set_b)), (1, (set_b, set_a))):
            c = 2 * ci + k
            issue_scd(c, s)

            @pl.when(c + 1 < NCHUNKS)
            def _():
                wait_idx(t)
                issue_gathers(c + 1, t)

            wait_gathers(s)

            @pl.when(c >= 1)
            def _():
                wait_scatter(t)

            compute(s)
            wait_scd(s)
            issue_scatter(s)

            @pl.when(c + 2 < NCHUNKS)
            def _():
                issue_idx(c + 2, s)

        return carry

    lax.fori_loop(0, NPAIRS, _pair, 0)

    wait_scatter(set_b)
    plsc.subcore_barrier()

    pltpu.async_copy(agg_sh.at[pl.ds(base_row, ROWS_PER_TILE)],
                     agg_out.at[cid, pl.ds(base_row, ROWS_PER_TILE)], sem_g_a)
    pltpu.async_copy(cnt_sh.at[pl.ds(base_row, ROWS_PER_TILE)],
                     cnt_out.at[cid, pl.ds(base_row, ROWS_PER_TILE)], sem_i_a)
    pltpu.make_async_copy(agg_sh.at[pl.ds(base_row, ROWS_PER_TILE)],
                          agg_out.at[cid, pl.ds(base_row, ROWS_PER_TILE)],
                          sem_g_a).wait()
    pltpu.make_async_copy(cnt_sh.at[pl.ds(base_row, ROWS_PER_TILE)],
                          cnt_out.at[cid, pl.ds(base_row, ROWS_PER_TILE)],
                          sem_i_a).wait()


_sc_edge = pl.kernel(
    _sc_edge_body,
    mesh=plsc.VectorSubcoreMesh(core_axis_name="c", subcore_axis_name="s"),
    compiler_params=pltpu.CompilerParams(use_tc_tiling_on_sc=False,
                                         needs_layout_passes=False),
    out_type=[
        jax.ShapeDtypeStruct((NC, N_PAD, D_OUT), jnp.float32),
        jax.ShapeDtypeStruct((NC, N_PAD, CW), jnp.float32),
    ],
    scratch_types=[
        pltpu.VMEM((CHUNK,), jnp.int32),
        pltpu.VMEM((CHUNK,), jnp.int32),
        pltpu.VMEM((CHUNK, D_OUT), jnp.float32),
        pltpu.VMEM((CHUNK, D_OUT), jnp.float32),
        pltpu.VMEM((CHUNK // 2, D_OUT), jnp.int32),
        pltpu.VMEM((CHUNK, D_OUT), jnp.float32),
        pltpu.VMEM((CHUNK,), jnp.int32),
        pltpu.VMEM((CHUNK,), jnp.int32),
        pltpu.VMEM((CHUNK, D_OUT), jnp.float32),
        pltpu.VMEM((CHUNK, D_OUT), jnp.float32),
        pltpu.VMEM((CHUNK // 2, D_OUT), jnp.int32),
        pltpu.VMEM((CHUNK, D_OUT), jnp.float32),
        pltpu.VMEM((CHUNK,), jnp.int32),
        pltpu.VMEM((CHUNK,), jnp.int32),
        pltpu.VMEM((CHUNK, CW), jnp.float32),
        pltpu.VMEM((CHUNK, CW), jnp.float32),
        pltpu.VMEM_SHARED((N_PAD, D_OUT), jnp.float32),
        pltpu.VMEM_SHARED((N_PAD, CW), jnp.float32),
        pltpu.SemaphoreType.DMA,
        pltpu.SemaphoreType.DMA,
        pltpu.SemaphoreType.DMA,
        pltpu.SemaphoreType.DMA,
        pltpu.SemaphoreType.DMA,
        pltpu.SemaphoreType.DMA,
        pltpu.SemaphoreType.DMA,
        pltpu.SemaphoreType.DMA,
    ],
)


def _out_body(a0_ref, a1_ref, c0_ref, c1_ref, w2_ref, b2_ref, o_ref):
    agg = a0_ref[...] + a1_ref[...]
    deg = c0_ref[...][:, :1] + c1_ref[...][:, :1]
    o_ref[...] = lax.dot_general(agg, w2_ref[...], _DOT,
                                 preferred_element_type=jnp.float32,
                                 precision=lax.Precision.HIGHEST) + deg * b2_ref[...]


_BO = 1000
_out_call = pl.pallas_call(
    _out_body,
    grid=(N_NODES // _BO,),
    in_specs=[
        pl.BlockSpec((_BO, D_OUT), lambda i: (i, 0)),
        pl.BlockSpec((_BO, D_OUT), lambda i: (i, 0)),
        pl.BlockSpec((_BO, CW), lambda i: (i, 0)),
        pl.BlockSpec((_BO, CW), lambda i: (i, 0)),
        pl.BlockSpec((D_OUT, D_OUT), lambda i: (0, 0)),
        pl.BlockSpec((1, D_OUT), lambda i: (0, 0)),
    ],
    out_specs=pl.BlockSpec((_BO, D_OUT), lambda i: (i, 0)),
    out_shape=jax.ShapeDtypeStruct((N_NODES, D_OUT), jnp.float32),
)


def kernel(node_feats, edge_index, edge_feats, W1, b1, W2, b2):
    src = edge_index[0].astype(jnp.int32)
    dst = edge_index[1].astype(jnp.int32)
    p, q = _pq_call(node_feats, W1[:D_NODE], W1[D_NODE:2 * D_NODE])
    e1 = _e1_call(edge_feats.reshape(N_EDGES // 2, 2 * D_EDGE),
                  W1[2 * D_NODE:], b1.reshape(1, D_OUT))
    agg2, cnt2 = _sc_edge(p, q, e1, src, dst)
    out = _out_call(agg2[0], agg2[1], cnt2[0], cnt2[1],
                    W2, b2.reshape(1, D_OUT))
    return out
